# Initial kernel scaffold; baseline (speedup 1.0000x reference)
import jax
import jax.numpy as jnp
from jax.experimental import pallas as pl


def kernel(label, feat, centers):
    raise NotImplementedError("write your pallas kernel here")


# speedup vs baseline: 1.2091x; 1.2091x over previous
import functools

import jax
import jax.numpy as jnp
from jax import lax
from jax.experimental import pallas as pl
from jax.experimental.pallas import tpu as pltpu
from jax.experimental.pallas import tpu_sc as plsc

BATCH = 16384
FEAT_DIM = 128
LANES = 16
GROUPS = FEAT_DIM // LANES

NUM_CORES = 2
NUM_SUBCORES = 16
NW = NUM_CORES * NUM_SUBCORES
ROWS_PER_W = BATCH // NW
CHUNK = 128
NCHUNK = ROWS_PER_W // CHUNK

_mesh = plsc.VectorSubcoreMesh(core_axis_name="c", subcore_axis_name="s")


@functools.partial(
    pl.kernel,
    mesh=_mesh,
    out_type=jax.ShapeDtypeStruct((NW, LANES), jnp.float32),
    scratch_types=[
        pltpu.VMEM((NCHUNK, CHUNK), jnp.int32),
        pltpu.VMEM((2, CHUNK, FEAT_DIM), jnp.float32),
        pltpu.VMEM((2, CHUNK, FEAT_DIM), jnp.float32),
        pltpu.VMEM((LANES,), jnp.float32),
        pltpu.SemaphoreType.DMA,
        pltpu.SemaphoreType.DMA,
        pltpu.SemaphoreType.DMA,
        pltpu.SemaphoreType.DMA,
    ],
)
def _center_loss_partials(label_hbm, feat_hbm, centers_hbm, out_hbm,
                          idx_v, cent_v, feat_v, acc_v,
                          sem_c0, sem_c1, sem_f0, sem_f1):
    wid = lax.axis_index("s") * NUM_CORES + lax.axis_index("c")
    base = wid * ROWS_PER_W

    pltpu.sync_copy(label_hbm.at[pl.ds(wid * NCHUNK, NCHUNK)], idx_v)

    sem_c = (sem_c0, sem_c1)
    sem_f = (sem_f0, sem_f1)

    def start(k, slot):
        pltpu.async_copy(centers_hbm.at[idx_v.at[k]], cent_v.at[slot],
                         sem_c[slot])
        pltpu.async_copy(feat_hbm.at[pl.ds(base + k * CHUNK, CHUNK)],
                         feat_v.at[slot], sem_f[slot])

    def wait(k, slot):
        pltpu.make_async_copy(centers_hbm.at[idx_v.at[k]], cent_v.at[slot],
                              sem_c[slot]).wait()
        pltpu.make_async_copy(feat_hbm.at[pl.ds(base + k * CHUNK, CHUNK)],
                              feat_v.at[slot], sem_f[slot]).wait()

    start(0, 0)

    accs = tuple(jnp.zeros((LANES,), jnp.float32) for _ in range(GROUPS))
    for k in range(NCHUNK):
        slot = k % 2
        wait(k, slot)
        if k + 1 < NCHUNK:
            start(k + 1, 1 - slot)

        def row_body(r, acc, _slot=slot):
            out = []
            for g in range(GROUPS):
                f = feat_v[_slot, r, pl.ds(g * LANES, LANES)]
                c = cent_v[_slot, r, pl.ds(g * LANES, LANES)]
                d = f - c
                out.append(acc[g] + d * d)
            return tuple(out)

        accs = lax.fori_loop(0, CHUNK, row_body, accs)

    total = accs[0]
    for g in range(1, GROUPS):
        total = total + accs[g]
    acc_v[...] = total
    pltpu.sync_copy(acc_v, out_hbm.at[wid])


def kernel(label, feat, centers):
    label2d = label.astype(jnp.int32).reshape(NW * NCHUNK, CHUNK)
    partials = _center_loss_partials(label2d, feat, centers)
    return jnp.sum(partials) * (0.5 / BATCH)


# speedup vs baseline: 1.2091x; 1.0000x over previous
import functools

import jax
import jax.numpy as jnp
from jax import lax
from jax.experimental import pallas as pl
from jax.experimental.pallas import tpu as pltpu
from jax.experimental.pallas import tpu_sc as plsc

BATCH = 16384
FEAT_DIM = 128
LANES = 16
GROUPS = FEAT_DIM // LANES

NUM_CORES = 2
NUM_SUBCORES = 16
NW = NUM_CORES * NUM_SUBCORES
ROWS_PER_W = BATCH // NW
CHUNK = 128
NCHUNK = ROWS_PER_W // CHUNK

_mesh = plsc.VectorSubcoreMesh(core_axis_name="c", subcore_axis_name="s")


@functools.partial(
    pl.kernel,
    mesh=_mesh,
    out_type=jax.ShapeDtypeStruct((NW, LANES), jnp.float32),
    scratch_types=[
        pltpu.VMEM((NCHUNK, CHUNK), jnp.int32),
        pltpu.VMEM((2, CHUNK, FEAT_DIM), jnp.float32),
        pltpu.VMEM((2, CHUNK, FEAT_DIM), jnp.float32),
        pltpu.VMEM((LANES,), jnp.float32),
        pltpu.SemaphoreType.DMA,
        pltpu.SemaphoreType.DMA,
        pltpu.SemaphoreType.DMA,
        pltpu.SemaphoreType.DMA,
    ],
)
def _center_loss_partials(label_hbm, feat_hbm, centers_hbm, out_hbm,
                          idx_v, cent_v, feat_v, acc_v,
                          sem_c0, sem_c1, sem_f0, sem_f1):
    wid = lax.axis_index("s") * NUM_CORES + lax.axis_index("c")
    base = wid * ROWS_PER_W

    pltpu.sync_copy(label_hbm.at[pl.ds(wid * NCHUNK, NCHUNK)], idx_v)

    sem_c = (sem_c0, sem_c1)
    sem_f = (sem_f0, sem_f1)

    def start(k, slot):
        pltpu.async_copy(centers_hbm.at[idx_v.at[k]], cent_v.at[slot],
                         sem_c[slot])
        pltpu.async_copy(feat_hbm.at[pl.ds(base + k * CHUNK, CHUNK)],
                         feat_v.at[slot], sem_f[slot])

    def wait(k, slot):
        pltpu.make_async_copy(centers_hbm.at[idx_v.at[k]], cent_v.at[slot],
                              sem_c[slot]).wait()
        pltpu.make_async_copy(feat_hbm.at[pl.ds(base + k * CHUNK, CHUNK)],
                              feat_v.at[slot], sem_f[slot]).wait()

    start(0, 0)

    accs = tuple(jnp.zeros((LANES,), jnp.float32) for _ in range(GROUPS))
    for k in range(NCHUNK):
        slot = k % 2
        wait(k, slot)
        if k + 1 < NCHUNK:
            start(k + 1, 1 - slot)

        def row_body(r, acc, _slot=slot):
            out = []
            for g in range(GROUPS):
                f = feat_v[_slot, r, pl.ds(g * LANES, LANES)]
                c = cent_v[_slot, r, pl.ds(g * LANES, LANES)]
                d = f - c
                out.append(acc[g] + d * d)
            return tuple(out)

        accs = plsc.parallel_loop(0, CHUNK, unroll=4, carry=accs)(row_body)

    total = accs[0]
    for g in range(1, GROUPS):
        total = total + accs[g]
    acc_v[...] = total
    pltpu.sync_copy(acc_v, out_hbm.at[wid])


def kernel(label, feat, centers):
    label2d = label.astype(jnp.int32).reshape(NW * NCHUNK, CHUNK)
    partials = _center_loss_partials(label2d, feat, centers)
    return jnp.sum(partials) * (0.5 / BATCH)
